# baseline (speedup 1.0000x reference)
import functools

import jax
import jax.numpy as jnp
from jax.experimental import pallas as pl
from jax.experimental.pallas import tpu as pltpu
from jax.experimental.pallas import tpu_sc as plsc

T = 4096
D = 768
DFF = 3072
E = 8
TOPK = 2
A = T * TOPK
B = 256
NP = A + E * B
NB = NP // B
DP = D // 2

_SC_CORES = 2
_SC_SUBCORES = 16
_NW = _SC_CORES * _SC_SUBCORES
_CHUNK = 32


def _pack_f32_to_bf16_pair(a):
    bits = jax.lax.bitcast_convert_type(a, jnp.int32)
    rnd = bits + 0x7FFF + jnp.bitwise_and(jax.lax.shift_right_logical(bits, 16), 1)
    top = jax.lax.shift_right_logical(rnd, 16)
    return jnp.bitwise_or(top[:, :DP], jax.lax.shift_left(top[:, DP:], 16))


def _unpack_bf16_pair_to_f32(v):
    lo = jax.lax.bitcast_convert_type(jax.lax.shift_left(v, 16), jnp.float32)
    hi = jax.lax.bitcast_convert_type(
        jnp.bitwise_and(v, jnp.int32(-65536)), jnp.float32)
    return jnp.concatenate([lo, hi], axis=1)


def _router_block(x_ref, wg_ref, bg_ref, wn_ref, bn_ref, noise_ref,
                  idx_ref, gate_ref, xpk_ref):
    x = x_ref[...]
    logits = jnp.dot(x, wg_ref[...], preferred_element_type=jnp.float32) + bg_ref[0]
    nlog = jnp.dot(x, wn_ref[...], preferred_element_type=jnp.float32) + bn_ref[0]
    noisy = logits + noise_ref[...] * jax.nn.softplus(nlog)

    lane = jax.lax.broadcasted_iota(jnp.int32, noisy.shape, 1)
    m1 = jnp.max(noisy, axis=1, keepdims=True)
    i1 = jnp.min(jnp.where(noisy == m1, lane, E), axis=1, keepdims=True)
    rest = jnp.where(lane == i1, -jnp.inf, noisy)
    m2 = jnp.max(rest, axis=1, keepdims=True)
    i2 = jnp.min(jnp.where(rest == m2, lane, E), axis=1, keepdims=True)
    g1 = 1.0 / (1.0 + jnp.exp(m2 - m1))

    lane2 = jax.lax.broadcasted_iota(jnp.int32, (noisy.shape[0], TOPK), 1)
    idx_ref[...] = jnp.where(lane2 == 0, i1, i2)
    gate_ref[...] = jnp.where(lane2 == 0, g1, 1.0 - g1)
    xpk_ref[...] = _pack_f32_to_bf16_pair(x)


def _sc_gather_rows(table, idx):
    n, d = idx.shape[0], table.shape[1]
    b_per_w = n // _NW
    chunk = _CHUNK
    nk = b_per_w // chunk

    @functools.partial(
        pl.kernel,
        mesh=plsc.VectorSubcoreMesh(core_axis_name="c", subcore_axis_name="s"),
        out_type=jax.ShapeDtypeStruct((n, d), table.dtype),
        scratch_types=[
            pltpu.VMEM((b_per_w,), jnp.int32),
            pltpu.VMEM((b_per_w, d), table.dtype),
            pltpu.SemaphoreType.DMA,
        ],
    )
    def k(table_hbm, idx_hbm, out_hbm, idx_v, rows_v, sem):
        wid = jax.lax.axis_index("s") * _SC_CORES + jax.lax.axis_index("c")
        base = wid * b_per_w
        pltpu.sync_copy(idx_hbm.at[pl.ds(base, b_per_w)], idx_v)
        copies = [
            pltpu.make_async_copy(
                table_hbm.at[idx_v.at[pl.ds(c * chunk, chunk)]],
                rows_v.at[pl.ds(c * chunk, chunk)],
                sem,
            )
            for c in range(nk)
        ]
        for cp in copies:
            cp.start()
        for cp in copies:
            cp.wait()
        pltpu.sync_copy(rows_v, out_hbm.at[pl.ds(base, b_per_w)])

    return k(table, idx)


def _sc_dispatch_rows(rows, idx_all):
    n, d = rows.shape
    r_per_w = n // _NW

    @functools.partial(
        pl.kernel,
        mesh=plsc.VectorSubcoreMesh(core_axis_name="c", subcore_axis_name="s"),
        out_type=jax.ShapeDtypeStruct((NP, d), rows.dtype),
        scratch_types=[
            pltpu.VMEM((1, r_per_w), jnp.int32),
            pltpu.VMEM((r_per_w, d), rows.dtype),
            pltpu.SemaphoreType.DMA,
        ],
    )
    def k(rows_hbm, idx_hbm, out_hbm, idx_v, rows_v, sem):
        wid = jax.lax.axis_index("s") * _SC_CORES + jax.lax.axis_index("c")
        base = wid * r_per_w
        pltpu.sync_copy(rows_hbm.at[pl.ds(base, r_per_w)], rows_v)
        for kk in range(TOPK):
            pltpu.sync_copy(idx_hbm.at[kk, wid], idx_v)
            pltpu.async_copy(rows_v, out_hbm.at[idx_v.at[0]], sem).wait()

    return k(rows, idx_all)


def _ffn_block(be_ref, flag_ref, xg_ref, w1_ref, b1_ref, w2_ref,
               b2_ref, yg_ref, w1bf, w2bf):
    j = pl.program_id(0)
    be = be_ref[j]

    @pl.when(flag_ref[j] == 1)
    def _():
        w1bf[...] = w1_ref[0].astype(jnp.bfloat16)
        w2bf[...] = w2_ref[0].astype(jnp.bfloat16)

    @pl.when(be < E)
    def _():
        xb = _unpack_bf16_pair_to_f32(xg_ref[...]).astype(jnp.bfloat16)
        h = jnp.dot(xb, w1bf[...], preferred_element_type=jnp.float32) + b1_ref[0]
        hb = jnp.maximum(h, 0.0).astype(jnp.bfloat16)
        y = jnp.dot(hb, w2bf[...], preferred_element_type=jnp.float32)
        yg_ref[...] = _pack_f32_to_bf16_pair(y + b2_ref[0])


def _combine_block(ya_ref, yb_ref, g_ref, out_ref):
    g0 = g_ref[:, :1]
    g1 = g_ref[:, 1:]
    out_ref[...] = (g0 * _unpack_bf16_pair_to_f32(ya_ref[...])
                    + g1 * _unpack_bf16_pair_to_f32(yb_ref[...]))


def kernel(x, Wg, bg, Wn, bn, W1, b1, W2, b2):
    base_noise = jax.random.normal(jax.random.key(42), (T, E), dtype=jnp.float32)

    idx, gates, xpk = pl.pallas_call(
        _router_block,
        grid=(T // 512,),
        in_specs=[
            pl.BlockSpec((512, D), lambda t: (t, 0)),
            pl.BlockSpec((D, E), lambda t: (0, 0)),
            pl.BlockSpec((1, E), lambda t: (0, 0)),
            pl.BlockSpec((D, E), lambda t: (0, 0)),
            pl.BlockSpec((1, E), lambda t: (0, 0)),
            pl.BlockSpec((512, E), lambda t: (t, 0)),
        ],
        out_specs=[
            pl.BlockSpec((512, TOPK), lambda t: (t, 0)),
            pl.BlockSpec((512, TOPK), lambda t: (t, 0)),
            pl.BlockSpec((512, DP), lambda t: (t, 0)),
        ],
        out_shape=[
            jax.ShapeDtypeStruct((T, TOPK), jnp.int32),
            jax.ShapeDtypeStruct((T, TOPK), jnp.float32),
            jax.ShapeDtypeStruct((T, DP), jnp.int32),
        ],
    )(x, Wg, bg[None, :], Wn, bn[None, :], base_noise)

    eid = idx.reshape(A)
    oh = (eid[:, None] == jnp.arange(E)[None, :]).astype(jnp.int32)
    counts = oh.sum(axis=0)
    padded = ((counts + B - 1) // B) * B
    start = jnp.concatenate([jnp.zeros((1,), jnp.int32),
                             jnp.cumsum(padded)[:-1].astype(jnp.int32)])
    rank = ((jnp.cumsum(oh, axis=0) - oh) * oh).sum(axis=1)
    dest = (oh * start[None, :]).sum(axis=1) + rank

    end_e = (start + padded).astype(jnp.int32)
    blk = jnp.arange(NB, dtype=jnp.int32) * B
    block_expert = (blk[:, None] >= end_e[None, :]).astype(jnp.int32).sum(axis=1)
    valid = block_expert < E
    be_clamped = jnp.minimum(block_expert, E - 1)
    prev = jnp.concatenate([jnp.full((1,), -1, jnp.int32), be_clamped[:-1]])
    cast_flag = ((be_clamped != prev) & valid).astype(jnp.int32)
    be_arr = jnp.where(valid, be_clamped, E).astype(jnp.int32)

    dest2 = dest.reshape(T, TOPK).astype(jnp.int32)
    idx_all = jnp.stack([
        dest2[:, 0].reshape(_NW, 1, T // _NW),
        dest2[:, 1].reshape(_NW, 1, T // _NW),
    ])
    xg = _sc_dispatch_rows(xpk, idx_all)

    yg = pl.pallas_call(
        _ffn_block,
        grid_spec=pltpu.PrefetchScalarGridSpec(
            num_scalar_prefetch=2,
            grid=(NB,),
            in_specs=[
                pl.BlockSpec((B, DP), lambda j, be, fl: (j, 0)),
                pl.BlockSpec((1, D, DFF), lambda j, be, fl: (jnp.minimum(be[j], E - 1), 0, 0)),
                pl.BlockSpec((1, 1, DFF), lambda j, be, fl: (jnp.minimum(be[j], E - 1), 0, 0)),
                pl.BlockSpec((1, DFF, D), lambda j, be, fl: (jnp.minimum(be[j], E - 1), 0, 0)),
                pl.BlockSpec((1, 1, D), lambda j, be, fl: (jnp.minimum(be[j], E - 1), 0, 0)),
            ],
            out_specs=pl.BlockSpec((B, DP), lambda j, be, fl: (j, 0)),
            scratch_shapes=[
                pltpu.VMEM((D, DFF), jnp.bfloat16),
                pltpu.VMEM((DFF, D), jnp.bfloat16),
            ],
        ),
        out_shape=jax.ShapeDtypeStruct((NP, DP), jnp.int32),
    )(be_arr, cast_flag, xg, W1, b1[:, None, :], W2, b2[:, None, :])

    dest_r = jnp.concatenate([dest2[:, 0], dest2[:, 1]])
    y2 = _sc_gather_rows(yg, dest_r)
    nt = T // 512
    out = pl.pallas_call(
        _combine_block,
        grid=(nt,),
        in_specs=[
            pl.BlockSpec((512, DP), lambda t: (t, 0)),
            pl.BlockSpec((512, DP), lambda t: (nt + t, 0)),
            pl.BlockSpec((512, TOPK), lambda t: (t, 0)),
        ],
        out_specs=pl.BlockSpec((512, D), lambda t: (t, 0)),
        out_shape=jax.ShapeDtypeStruct((T, D), jnp.float32),
    )(y2, y2, gates)
    return out
